# Initial kernel scaffold; baseline (speedup 1.0000x reference)
import jax
import jax.numpy as jnp
from jax.experimental import pallas as pl


def kernel(x, edge_index, W1, b1, W2, b2, W3, b3, Wres, bres):
    raise NotImplementedError("write your pallas kernel here")


# speedup vs baseline: 10.4956x; 10.4956x over previous
import functools

import jax
import jax.numpy as jnp
from jax import lax
from jax.experimental import pallas as pl
from jax.experimental.pallas import tpu as pltpu
from jax.experimental.pallas import tpu_sc as plsc

N = 10000
D = 256
H = 128
E = 160000
NP = 10240
EP = 163840
EPT = EP // 16
EPW = EP // 32
G = 128
NBM = EPT // G
NBD = EPW // G
RPT = NP // 16
RB = 512
NRB = NP // RB

_mesh = plsc.VectorSubcoreMesh(core_axis_name="c", subcore_axis_name="s")


@functools.partial(
    pl.kernel,
    out_type=jax.ShapeDtypeStruct((2, NP), jnp.float32),
    mesh=_mesh,
    scratch_types=[
        pltpu.VMEM_SHARED((NP,), jnp.float32),
        pltpu.VMEM((RPT,), jnp.float32),
        pltpu.VMEM((G,), jnp.float32),
        pltpu.VMEM((NBD, G), jnp.int32),
    ],
)
def _deg(dst_hbm, out_hbm, acc, buf, ones, idxv):
    c = lax.axis_index("c")
    s = lax.axis_index("s")
    w = c * 16 + s
    for i in range(RPT // 16):
        buf[pl.ds(i * 16, 16)] = jnp.zeros((16,), jnp.float32)
    for i in range(G // 16):
        ones[pl.ds(i * 16, 16)] = jnp.ones((16,), jnp.float32)
    pltpu.sync_copy(buf, acc.at[pl.ds(s * RPT, RPT)])
    pltpu.sync_copy(dst_hbm.at[w], idxv)
    plsc.subcore_barrier()

    @pl.loop(0, NBD)
    def _(j):
        pltpu.sync_copy(ones, acc.at[idxv.at[j]], add=True)

    plsc.subcore_barrier()
    pltpu.sync_copy(acc.at[pl.ds(s * RPT, RPT)], buf)
    pltpu.sync_copy(buf, out_hbm.at[c, pl.ds(s * RPT, RPT)])


@functools.partial(
    pl.kernel,
    out_type=jax.ShapeDtypeStruct((2 * NP, H), jnp.float32),
    mesh=_mesh,
    scratch_types=[
        pltpu.VMEM_SHARED((NP, H), jnp.float32),
        pltpu.VMEM((NBM, G), jnp.int32),
        pltpu.VMEM((NBM, G), jnp.int32),
        pltpu.VMEM((G, H), jnp.float32),
        pltpu.SemaphoreType.DMA,
    ],
)
def _msg(hs_hbm, src_hbm, dst_hbm, out_hbm, acc, srcv, dstv, rows, sem):
    c = lax.axis_index("c")
    s = lax.axis_index("s")
    for k in range(RPT // G):
        r0 = s * RPT + k * G
        pltpu.sync_copy(hs_hbm.at[pl.ds(c * NP + r0, G)], rows)
        pltpu.sync_copy(rows, acc.at[pl.ds(r0, G)])
    pltpu.sync_copy(src_hbm.at[c, s], srcv)
    pltpu.sync_copy(dst_hbm.at[s], dstv)
    plsc.subcore_barrier()

    @pl.loop(0, NBM)
    def _(j):
        pltpu.async_copy(hs_hbm.at[srcv.at[j]], rows, sem).wait()
        pltpu.sync_copy(rows, acc.at[dstv.at[j]], add=True)

    plsc.subcore_barrier()
    for k in range(RPT // G):
        r0 = s * RPT + k * G
        pltpu.sync_copy(acc.at[pl.ds(r0, G)], rows)
        pltpu.sync_copy(rows, out_hbm.at[pl.ds(c * NP + r0, G)])


def _dis_body(dga_ref, dgb_ref, out_ref):
    deg = dga_ref[...] + dgb_ref[...] + 1.0
    out_ref[...] = lax.rsqrt(deg)


_dis_call = pl.pallas_call(
    _dis_body,
    out_shape=jax.ShapeDtypeStruct((NP // G, G), jnp.float32),
)


def _k1_body(x_ref, w1_ref, wres_ref, bres_ref, dis_ref, hs_ref, res_ref):
    xb = x_ref[...]
    dis = dis_ref[...]
    hs_ref[...] = jnp.dot(xb, w1_ref[...], preferred_element_type=jnp.float32) * dis
    res_ref[...] = (
        jnp.dot(xb, wres_ref[...], preferred_element_type=jnp.float32) + bres_ref[...]
    )


_k1_call = pl.pallas_call(
    _k1_body,
    grid=(NRB, 2),
    in_specs=[
        pl.BlockSpec((RB, D), lambda i, h: (i, 0)),
        pl.BlockSpec((D, H), lambda i, h: (0, h)),
        pl.BlockSpec((D, H), lambda i, h: (0, h)),
        pl.BlockSpec((1, H), lambda i, h: (0, h)),
        pl.BlockSpec((RB, 1), lambda i, h: (i, 0)),
    ],
    out_specs=[
        pl.BlockSpec((RB, H), lambda i, h: (h * NRB + i, 0)),
        pl.BlockSpec((RB, H), lambda i, h: (i, h)),
    ],
    out_shape=[
        jax.ShapeDtypeStruct((2 * NP, H), jnp.float32),
        jax.ShapeDtypeStruct((NP, D), jnp.float32),
    ],
)


def _comb_body(s0_ref, s1_ref, dis_ref, b_ref, w_ref, hs_ref):
    dis = dis_ref[...]
    b = b_ref[...]
    t0 = jnp.maximum(dis * s0_ref[...] + b[:, :H], 0.0)
    t1 = jnp.maximum(dis * s1_ref[...] + b[:, H:], 0.0)
    wb = w_ref[...]
    acc = jnp.dot(t0, wb[:H, :], preferred_element_type=jnp.float32)
    acc = acc + jnp.dot(t1, wb[H:, :], preferred_element_type=jnp.float32)
    hs_ref[...] = acc * dis


_comb_call = pl.pallas_call(
    _comb_body,
    grid=(NRB, 2),
    in_specs=[
        pl.BlockSpec((RB, H), lambda i, h: (i, 0)),
        pl.BlockSpec((RB, H), lambda i, h: (NRB + i, 0)),
        pl.BlockSpec((RB, 1), lambda i, h: (i, 0)),
        pl.BlockSpec((1, D), lambda i, h: (0, 0)),
        pl.BlockSpec((D, H), lambda i, h: (0, h)),
    ],
    out_specs=pl.BlockSpec((RB, H), lambda i, h: (h * NRB + i, 0)),
    out_shape=jax.ShapeDtypeStruct((2 * NP, H), jnp.float32),
)


def _final_body(s_ref, dis_ref, b_ref, res_ref, out_ref):
    out_ref[...] = (
        jnp.maximum(dis_ref[...] * s_ref[...] + b_ref[...], 0.0) + res_ref[...]
    )


_final_call = pl.pallas_call(
    _final_body,
    grid=(NRB, 2),
    in_specs=[
        pl.BlockSpec((RB, H), lambda i, h: (h * NRB + i, 0)),
        pl.BlockSpec((RB, 1), lambda i, h: (i, 0)),
        pl.BlockSpec((1, H), lambda i, h: (0, h)),
        pl.BlockSpec((RB, H), lambda i, h: (i, h)),
    ],
    out_specs=pl.BlockSpec((RB, H), lambda i, h: (i, h)),
    out_shape=jax.ShapeDtypeStruct((NP, D), jnp.float32),
)


def kernel(x, edge_index, W1, b1, W2, b2, W3, b3, Wres, bres):
    src = edge_index[0].astype(jnp.int32)
    dst = edge_index[1].astype(jnp.int32)
    pad = EP - E
    padi = jnp.arange(pad, dtype=jnp.int32)
    srcp = jnp.concatenate([src, (padi * 101) % N])
    dstp = jnp.concatenate([dst, N + padi % (NP - N)])
    src2 = jnp.stack([srcp, srcp + NP]).reshape(2, 16, NBM, G)
    dstm = dstp.reshape(16, NBM, G)
    dstd = dstp.reshape(32, NBD, G)
    xpad = jnp.concatenate([x, jnp.zeros((NP - N, D), x.dtype)])

    degp = _deg(dstd)
    dis = _dis_call(degp[0].reshape(NP // G, G), degp[1].reshape(NP // G, G))
    dis_col = dis.reshape(NP, 1)

    b1r, b2r, b3r = b1.reshape(1, D), b2.reshape(1, D), b3.reshape(1, D)
    hs1, res = _k1_call(xpad, W1, Wres, bres.reshape(1, D), dis_col)
    s1 = _msg(hs1, src2, dstm)
    hs2 = _comb_call(s1, s1, dis_col, b1r, W2)
    s2 = _msg(hs2, src2, dstm)
    hs3 = _comb_call(s2, s2, dis_col, b2r, W3)
    s3 = _msg(hs3, src2, dstm)
    out = _final_call(s3, dis_col, b3r, res)
    return out[:N]


# speedup vs baseline: 13.8311x; 1.3178x over previous
import functools

import jax
import jax.numpy as jnp
from jax import lax
from jax.experimental import pallas as pl
from jax.experimental.pallas import tpu as pltpu
from jax.experimental.pallas import tpu_sc as plsc

N = 10000
D = 256
H = 128
E = 160000
NP = 10240
EP = 163840
EPT = EP // 16
EPW = EP // 32
G = 128
NBM = EPT // G
NBD = EPW // G
CH = 20
NCH = NBM // CH
RPT = NP // 16
RB = 512
NRB = NP // RB

_mesh = plsc.VectorSubcoreMesh(core_axis_name="c", subcore_axis_name="s")


@functools.partial(
    pl.kernel,
    out_type=jax.ShapeDtypeStruct((2, NP), jnp.float32),
    mesh=_mesh,
    scratch_types=[
        pltpu.VMEM_SHARED((NP,), jnp.float32),
        pltpu.VMEM((RPT,), jnp.float32),
        pltpu.VMEM((G,), jnp.float32),
        pltpu.VMEM((NBD, G), jnp.int32),
    ],
)
def _deg(dst_hbm, out_hbm, acc, buf, ones, idxv):
    c = lax.axis_index("c")
    s = lax.axis_index("s")
    w = c * 16 + s
    for i in range(RPT // 16):
        buf[pl.ds(i * 16, 16)] = jnp.zeros((16,), jnp.float32)
    for i in range(G // 16):
        ones[pl.ds(i * 16, 16)] = jnp.ones((16,), jnp.float32)
    pltpu.sync_copy(buf, acc.at[pl.ds(s * RPT, RPT)])
    pltpu.sync_copy(dst_hbm.at[w], idxv)
    plsc.subcore_barrier()

    @pl.loop(0, NBD)
    def _(j):
        pltpu.sync_copy(ones, acc.at[idxv.at[j]], add=True)

    plsc.subcore_barrier()
    pltpu.sync_copy(acc.at[pl.ds(s * RPT, RPT)], buf)
    pltpu.sync_copy(buf, out_hbm.at[c, pl.ds(s * RPT, RPT)])


@functools.partial(
    pl.kernel,
    out_type=jax.ShapeDtypeStruct((2 * NP, H), jnp.float32),
    mesh=_mesh,
    scratch_types=[
        pltpu.VMEM_SHARED((NP, H), jnp.float32),
        pltpu.VMEM((CH, G), jnp.int32),
        pltpu.VMEM((CH, G), jnp.int32),
        pltpu.VMEM((G, H), jnp.float32),
        pltpu.VMEM((G, H), jnp.float32),
        pltpu.SemaphoreType.DMA,
        pltpu.SemaphoreType.DMA,
    ],
)
def _msg(hs_hbm, src_hbm, dst_hbm, out_hbm, acc, srcv, dstv, rows0, rows1, sem0, sem1):
    c = lax.axis_index("c")
    s = lax.axis_index("s")
    for k in range(RPT // G):
        r0 = s * RPT + k * G
        pltpu.sync_copy(hs_hbm.at[pl.ds(c * NP + r0, G)], rows0)
        pltpu.sync_copy(rows0, acc.at[pl.ds(r0, G)])
    plsc.subcore_barrier()

    @pl.loop(0, NCH)
    def _(ch):
        pltpu.sync_copy(src_hbm.at[c, s, ch], srcv)
        pltpu.sync_copy(dst_hbm.at[s, ch], dstv)
        pltpu.async_copy(hs_hbm.at[srcv.at[0]], rows0, sem0)
        for b in range(0, CH, 2):
            pltpu.async_copy(hs_hbm.at[srcv.at[b + 1]], rows1, sem1)
            pltpu.make_async_copy(hs_hbm.at[srcv.at[b]], rows0, sem0).wait()
            pltpu.sync_copy(rows0, acc.at[dstv.at[b]], add=True)
            if b + 2 < CH:
                pltpu.async_copy(hs_hbm.at[srcv.at[b + 2]], rows0, sem0)
            pltpu.make_async_copy(hs_hbm.at[srcv.at[b + 1]], rows1, sem1).wait()
            pltpu.sync_copy(rows1, acc.at[dstv.at[b + 1]], add=True)

    plsc.subcore_barrier()
    for k in range(RPT // G):
        r0 = s * RPT + k * G
        pltpu.sync_copy(acc.at[pl.ds(r0, G)], rows0)
        pltpu.sync_copy(rows0, out_hbm.at[pl.ds(c * NP + r0, G)])


def _dis_body(dga_ref, dgb_ref, out_ref):
    deg = dga_ref[...] + dgb_ref[...] + 1.0
    out_ref[...] = lax.rsqrt(deg)


_dis_call = pl.pallas_call(
    _dis_body,
    out_shape=jax.ShapeDtypeStruct((NP // G, G), jnp.float32),
)


def _k1_body(x_ref, w1_ref, wres_ref, bres_ref, dis_ref, hs_ref, res_ref):
    xb = x_ref[...]
    dis = dis_ref[...]
    hs_ref[...] = jnp.dot(xb, w1_ref[...], preferred_element_type=jnp.float32) * dis
    res_ref[...] = (
        jnp.dot(xb, wres_ref[...], preferred_element_type=jnp.float32) + bres_ref[...]
    )


_k1_call = pl.pallas_call(
    _k1_body,
    grid=(NRB, 2),
    in_specs=[
        pl.BlockSpec((RB, D), lambda i, h: (i, 0)),
        pl.BlockSpec((D, H), lambda i, h: (0, h)),
        pl.BlockSpec((D, H), lambda i, h: (0, h)),
        pl.BlockSpec((1, H), lambda i, h: (0, h)),
        pl.BlockSpec((RB, 1), lambda i, h: (i, 0)),
    ],
    out_specs=[
        pl.BlockSpec((RB, H), lambda i, h: (h * NRB + i, 0)),
        pl.BlockSpec((RB, H), lambda i, h: (i, h)),
    ],
    out_shape=[
        jax.ShapeDtypeStruct((2 * NP, H), jnp.float32),
        jax.ShapeDtypeStruct((NP, D), jnp.float32),
    ],
)


def _comb_body(s0_ref, s1_ref, dis_ref, b_ref, w_ref, hs_ref):
    dis = dis_ref[...]
    b = b_ref[...]
    t0 = jnp.maximum(dis * s0_ref[...] + b[:, :H], 0.0)
    t1 = jnp.maximum(dis * s1_ref[...] + b[:, H:], 0.0)
    wb = w_ref[...]
    acc = jnp.dot(t0, wb[:H, :], preferred_element_type=jnp.float32)
    acc = acc + jnp.dot(t1, wb[H:, :], preferred_element_type=jnp.float32)
    hs_ref[...] = acc * dis


_comb_call = pl.pallas_call(
    _comb_body,
    grid=(NRB, 2),
    in_specs=[
        pl.BlockSpec((RB, H), lambda i, h: (i, 0)),
        pl.BlockSpec((RB, H), lambda i, h: (NRB + i, 0)),
        pl.BlockSpec((RB, 1), lambda i, h: (i, 0)),
        pl.BlockSpec((1, D), lambda i, h: (0, 0)),
        pl.BlockSpec((D, H), lambda i, h: (0, h)),
    ],
    out_specs=pl.BlockSpec((RB, H), lambda i, h: (h * NRB + i, 0)),
    out_shape=jax.ShapeDtypeStruct((2 * NP, H), jnp.float32),
)


def _final_body(s_ref, dis_ref, b_ref, res_ref, out_ref):
    out_ref[...] = (
        jnp.maximum(dis_ref[...] * s_ref[...] + b_ref[...], 0.0) + res_ref[...]
    )


_final_call = pl.pallas_call(
    _final_body,
    grid=(NRB, 2),
    in_specs=[
        pl.BlockSpec((RB, H), lambda i, h: (h * NRB + i, 0)),
        pl.BlockSpec((RB, 1), lambda i, h: (i, 0)),
        pl.BlockSpec((1, H), lambda i, h: (0, h)),
        pl.BlockSpec((RB, H), lambda i, h: (i, h)),
    ],
    out_specs=pl.BlockSpec((RB, H), lambda i, h: (i, h)),
    out_shape=jax.ShapeDtypeStruct((NP, D), jnp.float32),
)


def kernel(x, edge_index, W1, b1, W2, b2, W3, b3, Wres, bres):
    src = edge_index[0].astype(jnp.int32)
    dst = edge_index[1].astype(jnp.int32)
    pad = EP - E
    padi = jnp.arange(pad, dtype=jnp.int32)
    srcp = jnp.concatenate([src, (padi * 101) % N])
    dstp = jnp.concatenate([dst, N + padi % (NP - N)])
    src2 = jnp.stack([srcp, srcp + NP]).reshape(2, 16, NCH, CH, G)
    dstm = dstp.reshape(16, NCH, CH, G)
    dstd = dstp.reshape(32, NBD, G)
    xpad = jnp.concatenate([x, jnp.zeros((NP - N, D), x.dtype)])

    degp = _deg(dstd)
    dis = _dis_call(degp[0].reshape(NP // G, G), degp[1].reshape(NP // G, G))
    dis_col = dis.reshape(NP, 1)

    b1r, b2r, b3r = b1.reshape(1, D), b2.reshape(1, D), b3.reshape(1, D)
    hs1, res = _k1_call(xpad, W1, Wres, bres.reshape(1, D), dis_col)
    s1 = _msg(hs1, src2, dstm)
    hs2 = _comb_call(s1, s1, dis_col, b1r, W2)
    s2 = _msg(hs2, src2, dstm)
    hs3 = _comb_call(s2, s2, dis_col, b2r, W3)
    s3 = _msg(hs3, src2, dstm)
    out = _final_call(s3, dis_col, b3r, res)
    return out[:N]


# speedup vs baseline: 13.9256x; 1.0068x over previous
import functools

import jax
import jax.numpy as jnp
from jax import lax
from jax.experimental import pallas as pl
from jax.experimental.pallas import tpu as pltpu
from jax.experimental.pallas import tpu_sc as plsc

N = 10000
D = 256
H = 128
E = 160000
NP = 10240
EP = 163840
EPT = EP // 16
EPW = EP // 32
G = 128
NBM = EPT // G
NBD = EPW // G
CH = 20
NCH = NBM // CH
RPT = NP // 16
RB = 512
NRB = NP // RB

_mesh = plsc.VectorSubcoreMesh(core_axis_name="c", subcore_axis_name="s")


@functools.partial(
    pl.kernel,
    out_type=jax.ShapeDtypeStruct((2, NP), jnp.float32),
    mesh=_mesh,
    scratch_types=[
        pltpu.VMEM_SHARED((NP,), jnp.float32),
        pltpu.VMEM((RPT,), jnp.float32),
        pltpu.VMEM((G,), jnp.float32),
        pltpu.VMEM((NBD, G), jnp.int32),
    ],
)
def _deg(dst_hbm, out_hbm, acc, buf, ones, idxv):
    c = lax.axis_index("c")
    s = lax.axis_index("s")
    w = c * 16 + s
    for i in range(RPT // 16):
        buf[pl.ds(i * 16, 16)] = jnp.zeros((16,), jnp.float32)
    for i in range(G // 16):
        ones[pl.ds(i * 16, 16)] = jnp.ones((16,), jnp.float32)
    pltpu.sync_copy(buf, acc.at[pl.ds(s * RPT, RPT)])
    pltpu.sync_copy(dst_hbm.at[w], idxv)
    plsc.subcore_barrier()

    @pl.loop(0, NBD)
    def _(j):
        pltpu.sync_copy(ones, acc.at[idxv.at[j]], add=True)

    plsc.subcore_barrier()
    pltpu.sync_copy(acc.at[pl.ds(s * RPT, RPT)], buf)
    pltpu.sync_copy(buf, out_hbm.at[c, pl.ds(s * RPT, RPT)])


@functools.partial(
    pl.kernel,
    out_type=jax.ShapeDtypeStruct((2 * NP, H), jnp.float32),
    mesh=_mesh,
    scratch_types=[
        pltpu.VMEM_SHARED((NP, H), jnp.float32),
        pltpu.VMEM((CH, G), jnp.int32),
        pltpu.VMEM((CH, G), jnp.int32),
        pltpu.VMEM((G, H), jnp.float32),
        pltpu.VMEM((G, H), jnp.float32),
        pltpu.SemaphoreType.DMA,
        pltpu.SemaphoreType.DMA,
    ],
)
def _msg(hs_hbm, src_hbm, dst_hbm, out_hbm, acc, srcv, dstv, rows0, rows1, sem0, sem1):
    c = lax.axis_index("c")
    s = lax.axis_index("s")
    base = c * NP + s * RPT
    nci = RPT // G
    pltpu.async_copy(hs_hbm.at[pl.ds(base, G)], rows0, sem0)
    for k in range(nci):
        if k + 1 < nci:
            nxt, nsem = (rows1, sem1) if k % 2 == 0 else (rows0, sem0)
            pltpu.async_copy(hs_hbm.at[pl.ds(base + (k + 1) * G, G)], nxt, nsem)
        cur, csem = (rows0, sem0) if k % 2 == 0 else (rows1, sem1)
        pltpu.make_async_copy(hs_hbm.at[pl.ds(base + k * G, G)], cur, csem).wait()
        pltpu.sync_copy(cur, acc.at[pl.ds(s * RPT + k * G, G)])
    plsc.subcore_barrier()

    @pl.loop(0, NCH)
    def _(ch):
        pltpu.sync_copy(src_hbm.at[c, s, ch], srcv)
        pltpu.sync_copy(dst_hbm.at[s, ch], dstv)
        pltpu.async_copy(hs_hbm.at[srcv.at[0]], rows0, sem0)
        for b in range(0, CH, 2):
            pltpu.async_copy(hs_hbm.at[srcv.at[b + 1]], rows1, sem1)
            pltpu.make_async_copy(hs_hbm.at[srcv.at[b]], rows0, sem0).wait()
            pltpu.sync_copy(rows0, acc.at[dstv.at[b]], add=True)
            if b + 2 < CH:
                pltpu.async_copy(hs_hbm.at[srcv.at[b + 2]], rows0, sem0)
            pltpu.make_async_copy(hs_hbm.at[srcv.at[b + 1]], rows1, sem1).wait()
            pltpu.sync_copy(rows1, acc.at[dstv.at[b + 1]], add=True)

    plsc.subcore_barrier()
    pltpu.async_copy(acc.at[pl.ds(s * RPT, G)], rows0, sem0)
    for k in range(nci):
        if k + 1 < nci:
            nxt, nsem = (rows1, sem1) if k % 2 == 0 else (rows0, sem0)
            pltpu.async_copy(acc.at[pl.ds(s * RPT + (k + 1) * G, G)], nxt, nsem)
        cur, csem = (rows0, sem0) if k % 2 == 0 else (rows1, sem1)
        pltpu.make_async_copy(acc.at[pl.ds(s * RPT + k * G, G)], cur, csem).wait()
        pltpu.sync_copy(cur, out_hbm.at[pl.ds(base + k * G, G)])


def _k1_body(x_ref, w1_ref, wres_ref, bres_ref, d0_ref, d1_ref, hs_ref, res_ref):
    xb = x_ref[...]
    dis = lax.rsqrt(d0_ref[...] + d1_ref[...] + 1.0)
    hs_ref[...] = jnp.dot(xb, w1_ref[...], preferred_element_type=jnp.float32) * dis
    res_ref[...] = (
        jnp.dot(xb, wres_ref[...], preferred_element_type=jnp.float32) + bres_ref[...]
    )


_k1_call = pl.pallas_call(
    _k1_body,
    grid=(NRB, 2),
    in_specs=[
        pl.BlockSpec((RB, D), lambda i, h: (i, 0)),
        pl.BlockSpec((D, H), lambda i, h: (0, h)),
        pl.BlockSpec((D, H), lambda i, h: (0, h)),
        pl.BlockSpec((1, H), lambda i, h: (0, h)),
        pl.BlockSpec((RB, 1), lambda i, h: (i, 0)),
        pl.BlockSpec((RB, 1), lambda i, h: (i, 0)),
    ],
    out_specs=[
        pl.BlockSpec((RB, H), lambda i, h: (h * NRB + i, 0)),
        pl.BlockSpec((RB, H), lambda i, h: (i, h)),
    ],
    out_shape=[
        jax.ShapeDtypeStruct((2 * NP, H), jnp.float32),
        jax.ShapeDtypeStruct((NP, D), jnp.float32),
    ],
)


def _comb_body(s0_ref, s1_ref, d0_ref, d1_ref, b_ref, w_ref, hs_ref):
    dis = lax.rsqrt(d0_ref[...] + d1_ref[...] + 1.0)
    b = b_ref[...]
    t0 = jnp.maximum(dis * s0_ref[...] + b[:, :H], 0.0)
    t1 = jnp.maximum(dis * s1_ref[...] + b[:, H:], 0.0)
    wb = w_ref[...]
    acc = jnp.dot(t0, wb[:H, :], preferred_element_type=jnp.float32)
    acc = acc + jnp.dot(t1, wb[H:, :], preferred_element_type=jnp.float32)
    hs_ref[...] = acc * dis


_comb_call = pl.pallas_call(
    _comb_body,
    grid=(NRB, 2),
    in_specs=[
        pl.BlockSpec((RB, H), lambda i, h: (i, 0)),
        pl.BlockSpec((RB, H), lambda i, h: (NRB + i, 0)),
        pl.BlockSpec((RB, 1), lambda i, h: (i, 0)),
        pl.BlockSpec((RB, 1), lambda i, h: (i, 0)),
        pl.BlockSpec((1, D), lambda i, h: (0, 0)),
        pl.BlockSpec((D, H), lambda i, h: (0, h)),
    ],
    out_specs=pl.BlockSpec((RB, H), lambda i, h: (h * NRB + i, 0)),
    out_shape=jax.ShapeDtypeStruct((2 * NP, H), jnp.float32),
)


def _final_body(s_ref, d0_ref, d1_ref, b_ref, res_ref, out_ref):
    dis = lax.rsqrt(d0_ref[...] + d1_ref[...] + 1.0)
    out_ref[...] = jnp.maximum(dis * s_ref[...] + b_ref[...], 0.0) + res_ref[...]


_final_call = pl.pallas_call(
    _final_body,
    grid=(NRB, 2),
    in_specs=[
        pl.BlockSpec((RB, H), lambda i, h: (h * NRB + i, 0)),
        pl.BlockSpec((RB, 1), lambda i, h: (i, 0)),
        pl.BlockSpec((RB, 1), lambda i, h: (i, 0)),
        pl.BlockSpec((1, H), lambda i, h: (0, h)),
        pl.BlockSpec((RB, H), lambda i, h: (i, h)),
    ],
    out_specs=pl.BlockSpec((RB, H), lambda i, h: (i, h)),
    out_shape=jax.ShapeDtypeStruct((NP, D), jnp.float32),
)


def kernel(x, edge_index, W1, b1, W2, b2, W3, b3, Wres, bres):
    src = edge_index[0].astype(jnp.int32)
    dst = edge_index[1].astype(jnp.int32)
    pad = EP - E
    padi = jnp.arange(pad, dtype=jnp.int32)
    srcp = jnp.concatenate([src, (padi * 101) % N])
    dstp = jnp.concatenate([dst, N + padi % (NP - N)])
    src2 = jnp.stack([srcp, srcp + NP]).reshape(2, 16, NCH, CH, G)
    dstm = dstp.reshape(16, NCH, CH, G)
    dstd = dstp.reshape(32, NBD, G)
    xpad = jnp.concatenate([x, jnp.zeros((NP - N, D), x.dtype)])

    degp = _deg(dstd)
    d0c = degp[0].reshape(NP, 1)
    d1c = degp[1].reshape(NP, 1)

    b1r, b2r, b3r = b1.reshape(1, D), b2.reshape(1, D), b3.reshape(1, D)
    hs1, res = _k1_call(xpad, W1, Wres, bres.reshape(1, D), d0c, d1c)
    s1 = _msg(hs1, src2, dstm)
    hs2 = _comb_call(s1, s1, d0c, d1c, b1r, W2)
    s2 = _msg(hs2, src2, dstm)
    hs3 = _comb_call(s2, s2, d0c, d1c, b2r, W3)
    s3 = _msg(hs3, src2, dstm)
    out = _final_call(s3, d0c, d1c, b3r, res)
    return out[:N]


# speedup vs baseline: 15.3687x; 1.1036x over previous
import functools

import jax
import jax.numpy as jnp
from jax import lax
from jax.experimental import pallas as pl
from jax.experimental.pallas import tpu as pltpu
from jax.experimental.pallas import tpu_sc as plsc

N = 10000
D = 256
H = 128
E = 160000
NP = 10240
EP = 163840
EPT = EP // 16
EPW = EP // 32
G = 128
NBM = EPT // G
NBD = EPW // G
CH = 20
NCH = NBM // CH
RPT = NP // 16
RB = 1024
NRB = NP // RB

_mesh = plsc.VectorSubcoreMesh(core_axis_name="c", subcore_axis_name="s")


@functools.partial(
    pl.kernel,
    out_type=jax.ShapeDtypeStruct((2, NP), jnp.float32),
    mesh=_mesh,
    scratch_types=[
        pltpu.VMEM_SHARED((NP,), jnp.float32),
        pltpu.VMEM((RPT,), jnp.float32),
        pltpu.VMEM((G,), jnp.float32),
        pltpu.VMEM((NBD, G), jnp.int32),
    ],
)
def _deg(dst_hbm, out_hbm, acc, buf, ones, idxv):
    c = lax.axis_index("c")
    s = lax.axis_index("s")
    w = c * 16 + s
    for i in range(RPT // 16):
        buf[pl.ds(i * 16, 16)] = jnp.zeros((16,), jnp.float32)
    for i in range(G // 16):
        ones[pl.ds(i * 16, 16)] = jnp.ones((16,), jnp.float32)
    pltpu.sync_copy(buf, acc.at[pl.ds(s * RPT, RPT)])
    pltpu.sync_copy(dst_hbm.at[w], idxv)
    plsc.subcore_barrier()

    @pl.loop(0, NBD)
    def _(j):
        pltpu.sync_copy(ones, acc.at[idxv.at[j]], add=True)

    plsc.subcore_barrier()
    pltpu.sync_copy(acc.at[pl.ds(s * RPT, RPT)], buf)
    pltpu.sync_copy(buf, out_hbm.at[c, pl.ds(s * RPT, RPT)])


@functools.partial(
    pl.kernel,
    out_type=jax.ShapeDtypeStruct((2 * NP, H), jnp.float32),
    mesh=_mesh,
    scratch_types=[
        pltpu.VMEM_SHARED((NP, H), jnp.float32),
        pltpu.VMEM((CH, G), jnp.int32),
        pltpu.VMEM((CH, G), jnp.int32),
        pltpu.VMEM((G, H), jnp.float32),
        pltpu.VMEM((G, H), jnp.float32),
        pltpu.SemaphoreType.DMA,
        pltpu.SemaphoreType.DMA,
    ],
)
def _msg(hs_hbm, src_hbm, dst_hbm, out_hbm, acc, srcv, dstv, rows0, rows1, sem0, sem1):
    c = lax.axis_index("c")
    s = lax.axis_index("s")
    base = c * NP + s * RPT
    nci = RPT // G
    pltpu.async_copy(hs_hbm.at[pl.ds(base, G)], rows0, sem0)
    for k in range(nci):
        if k + 1 < nci:
            nxt, nsem = (rows1, sem1) if k % 2 == 0 else (rows0, sem0)
            pltpu.async_copy(hs_hbm.at[pl.ds(base + (k + 1) * G, G)], nxt, nsem)
        cur, csem = (rows0, sem0) if k % 2 == 0 else (rows1, sem1)
        pltpu.make_async_copy(hs_hbm.at[pl.ds(base + k * G, G)], cur, csem).wait()
        pltpu.sync_copy(cur, acc.at[pl.ds(s * RPT + k * G, G)])
    plsc.subcore_barrier()

    @pl.loop(0, NCH)
    def _(ch):
        pltpu.sync_copy(src_hbm.at[c, s, ch], srcv)
        pltpu.sync_copy(dst_hbm.at[s, ch], dstv)
        pltpu.async_copy(hs_hbm.at[srcv.at[0]], rows0, sem0)
        for b in range(0, CH, 2):
            pltpu.async_copy(hs_hbm.at[srcv.at[b + 1]], rows1, sem1)
            pltpu.make_async_copy(hs_hbm.at[srcv.at[b]], rows0, sem0).wait()
            pltpu.sync_copy(rows0, acc.at[dstv.at[b]], add=True)
            if b + 2 < CH:
                pltpu.async_copy(hs_hbm.at[srcv.at[b + 2]], rows0, sem0)
            pltpu.make_async_copy(hs_hbm.at[srcv.at[b + 1]], rows1, sem1).wait()
            pltpu.sync_copy(rows1, acc.at[dstv.at[b + 1]], add=True)

    plsc.subcore_barrier()
    pltpu.async_copy(acc.at[pl.ds(s * RPT, G)], rows0, sem0)
    for k in range(nci):
        if k + 1 < nci:
            nxt, nsem = (rows1, sem1) if k % 2 == 0 else (rows0, sem0)
            pltpu.async_copy(acc.at[pl.ds(s * RPT + (k + 1) * G, G)], nxt, nsem)
        cur, csem = (rows0, sem0) if k % 2 == 0 else (rows1, sem1)
        pltpu.make_async_copy(acc.at[pl.ds(s * RPT + k * G, G)], cur, csem).wait()
        pltpu.sync_copy(cur, out_hbm.at[pl.ds(base + k * G, G)])


def _k1_body(x_ref, w1_ref, wres_ref, bres_ref, d0_ref, d1_ref, hs_ref, res_ref):
    xb = x_ref[...]
    dis = lax.rsqrt(d0_ref[...] + d1_ref[...] + 1.0)
    hs_ref[...] = jnp.dot(xb, w1_ref[...], preferred_element_type=jnp.float32) * dis
    res_ref[...] = (
        jnp.dot(xb, wres_ref[...], preferred_element_type=jnp.float32) + bres_ref[...]
    )


_k1_call = pl.pallas_call(
    _k1_body,
    grid=(2, NRB),
    in_specs=[
        pl.BlockSpec((RB, D), lambda h, i: (i, 0)),
        pl.BlockSpec((D, H), lambda h, i: (0, h)),
        pl.BlockSpec((D, H), lambda h, i: (0, h)),
        pl.BlockSpec((1, H), lambda h, i: (0, h)),
        pl.BlockSpec((RB, 1), lambda h, i: (i, 0)),
        pl.BlockSpec((RB, 1), lambda h, i: (i, 0)),
    ],
    out_specs=[
        pl.BlockSpec((RB, H), lambda h, i: (h * NRB + i, 0)),
        pl.BlockSpec((RB, H), lambda h, i: (i, h)),
    ],
    out_shape=[
        jax.ShapeDtypeStruct((2 * NP, H), jnp.float32),
        jax.ShapeDtypeStruct((NP, D), jnp.float32),
    ],
)


def _comb_body(s0_ref, s1_ref, d0_ref, d1_ref, b_ref, w_ref, hs_ref):
    dis = lax.rsqrt(d0_ref[...] + d1_ref[...] + 1.0)
    b = b_ref[...]
    t0 = jnp.maximum(dis * s0_ref[...] + b[:, :H], 0.0)
    t1 = jnp.maximum(dis * s1_ref[...] + b[:, H:], 0.0)
    wb = w_ref[...]
    acc = jnp.dot(t0, wb[:H, :], preferred_element_type=jnp.float32)
    acc = acc + jnp.dot(t1, wb[H:, :], preferred_element_type=jnp.float32)
    hs_ref[...] = acc * dis


_comb_call = pl.pallas_call(
    _comb_body,
    grid=(2, NRB),
    in_specs=[
        pl.BlockSpec((RB, H), lambda h, i: (i, 0)),
        pl.BlockSpec((RB, H), lambda h, i: (NRB + i, 0)),
        pl.BlockSpec((RB, 1), lambda h, i: (i, 0)),
        pl.BlockSpec((RB, 1), lambda h, i: (i, 0)),
        pl.BlockSpec((1, D), lambda h, i: (0, 0)),
        pl.BlockSpec((D, H), lambda h, i: (0, h)),
    ],
    out_specs=pl.BlockSpec((RB, H), lambda h, i: (h * NRB + i, 0)),
    out_shape=jax.ShapeDtypeStruct((2 * NP, H), jnp.float32),
)


def _final_body(s_ref, d0_ref, d1_ref, b_ref, res_ref, out_ref):
    dis = lax.rsqrt(d0_ref[...] + d1_ref[...] + 1.0)
    out_ref[...] = jnp.maximum(dis * s_ref[...] + b_ref[...], 0.0) + res_ref[...]


_final_call = pl.pallas_call(
    _final_body,
    grid=(2, NRB),
    in_specs=[
        pl.BlockSpec((RB, H), lambda h, i: (h * NRB + i, 0)),
        pl.BlockSpec((RB, 1), lambda h, i: (i, 0)),
        pl.BlockSpec((RB, 1), lambda h, i: (i, 0)),
        pl.BlockSpec((1, H), lambda h, i: (0, h)),
        pl.BlockSpec((RB, H), lambda h, i: (i, h)),
    ],
    out_specs=pl.BlockSpec((RB, H), lambda h, i: (i, h)),
    out_shape=jax.ShapeDtypeStruct((NP, D), jnp.float32),
)


def kernel(x, edge_index, W1, b1, W2, b2, W3, b3, Wres, bres):
    src = edge_index[0].astype(jnp.int32)
    dst = edge_index[1].astype(jnp.int32)
    pad = EP - E
    padi = jnp.arange(pad, dtype=jnp.int32)
    srcp = jnp.concatenate([src, (padi * 101) % N])
    dstp = jnp.concatenate([dst, N + padi % (NP - N)])
    src2 = jnp.stack([srcp, srcp + NP]).reshape(2, 16, NCH, CH, G)
    dstm = dstp.reshape(16, NCH, CH, G)
    dstd = dstp.reshape(32, NBD, G)
    xpad = jnp.concatenate([x, jnp.zeros((NP - N, D), x.dtype)])

    degp = _deg(dstd)
    d0c = degp[0].reshape(NP, 1)
    d1c = degp[1].reshape(NP, 1)

    b1r, b2r, b3r = b1.reshape(1, D), b2.reshape(1, D), b3.reshape(1, D)
    hs1, res = _k1_call(xpad, W1, Wres, bres.reshape(1, D), d0c, d1c)
    s1 = _msg(hs1, src2, dstm)
    hs2 = _comb_call(s1, s1, d0c, d1c, b1r, W2)
    s2 = _msg(hs2, src2, dstm)
    hs3 = _comb_call(s2, s2, d0c, d1c, b2r, W3)
    s3 = _msg(hs3, src2, dstm)
    out = _final_call(s3, d0c, d1c, b3r, res)
    return out[:N]
